# Initial kernel scaffold; baseline (speedup 1.0000x reference)
import functools

import jax
import jax.numpy as jnp
from jax import lax
from jax.experimental import pallas as pl
from jax.experimental.pallas import tpu as pltpu
from jax.experimental.pallas import tpu_sc as plsc

N = 10000
NP = 10240
F_IN = 128
H = 32
G = 128
E = 320000
NC = 2
NS = 16
NW = NC * NS
CH = 128
EPT = 10112
NCHUNK = EPT // CH
EP = EPT * NW
RPT = NP // NS
DUMP = 10200
NB = 5
BR = NP // NB

_mesh = plsc.VectorSubcoreMesh(core_axis_name="c", subcore_axis_name="s")

_f32 = jnp.float32
_i32 = jnp.int32


@functools.partial(
    pl.kernel,
    out_type=jax.ShapeDtypeStruct((NC, NP), _f32),
    mesh=_mesh,
    scratch_types=[
        pltpu.VMEM((CH,), _i32),
        pltpu.VMEM((CH,), _f32),
        pltpu.VMEM((RPT,), _f32),
        pltpu.VMEM_SHARED((NP,), _f32),
    ],
)
def _sc_deg(dst_hbm, out_hbm, didx, ones, zbuf, acc):
    c = lax.axis_index("c")
    s = lax.axis_index("s")
    wid = s * NC + c

    def fill(i, _):
        zbuf[pl.ds(i * 16, 16)] = jnp.zeros((16,), _f32)
        return 0

    lax.fori_loop(0, RPT // 16, fill, 0)

    def fill1(i, _):
        ones[pl.ds(i * 16, 16)] = jnp.ones((16,), _f32)
        return 0

    lax.fori_loop(0, CH // 16, fill1, 0)
    pltpu.sync_copy(zbuf, acc.at[pl.ds(s * RPT, RPT)])
    plsc.subcore_barrier()

    def chunk(j, _):
        off = wid * EPT + j * CH
        pltpu.sync_copy(dst_hbm.at[pl.ds(off, CH)], didx)
        pltpu.sync_copy(ones, acc.at[didx], add=True)
        return 0

    lax.fori_loop(0, NCHUNK, chunk, 0)
    plsc.subcore_barrier()
    pltpu.sync_copy(acc.at[pl.ds(s * RPT, RPT)], out_hbm.at[c, pl.ds(s * RPT, RPT)])


@functools.partial(
    pl.kernel,
    out_type=jax.ShapeDtypeStruct((NC, NP, H), _f32),
    mesh=_mesh,
    scratch_types=[
        pltpu.VMEM((CH,), _i32),
        pltpu.VMEM((CH,), _i32),
        pltpu.VMEM((CH, H), _f32),
        pltpu.VMEM((RPT, H), _f32),
        pltpu.VMEM_SHARED((NP, H), _f32),
        pltpu.SemaphoreType.DMA,
    ],
)
def _sc_agg(y_hbm, src_hbm, dst_hbm, out_hbm, sidx, didx, rows, zbuf, acc, sem):
    c = lax.axis_index("c")
    s = lax.axis_index("s")
    wid = s * NC + c

    def fill(i, _):
        zbuf[i, pl.ds(0, 16)] = jnp.zeros((16,), _f32)
        zbuf[i, pl.ds(16, 16)] = jnp.zeros((16,), _f32)
        return 0

    lax.fori_loop(0, RPT, fill, 0)
    pltpu.sync_copy(zbuf, acc.at[pl.ds(s * RPT, RPT)])
    plsc.subcore_barrier()

    def chunk(j, _):
        off = wid * EPT + j * CH
        pltpu.sync_copy(src_hbm.at[pl.ds(off, CH)], sidx)
        pltpu.async_copy(y_hbm.at[sidx], rows, sem).wait()
        pltpu.sync_copy(dst_hbm.at[pl.ds(off, CH)], didx)
        pltpu.sync_copy(rows, acc.at[didx], add=True)
        return 0

    lax.fori_loop(0, NCHUNK, chunk, 0)
    plsc.subcore_barrier()
    pltpu.sync_copy(acc.at[pl.ds(s * RPT, RPT)],
                    out_hbm.at[c, pl.ds(s * RPT, RPT)])


def _tc1_body(x_ref, d_ref, w_ref, y_ref, dinv_ref):
    deg = d_ref[0] + d_ref[1] + 1.0
    dinv = lax.rsqrt(deg)
    y = jnp.dot(x_ref[...], w_ref[...], preferred_element_type=_f32)
    y_ref[...] = y * dinv
    dinv_ref[...] = dinv


def _tc1(xp, deg2, W1):
    return pl.pallas_call(
        _tc1_body,
        grid=(NB,),
        in_specs=[
            pl.BlockSpec((BR, F_IN), lambda i: (i, 0)),
            pl.BlockSpec((NC, BR, 1), lambda i: (0, i, 0)),
            pl.BlockSpec((F_IN, H), lambda i: (0, 0)),
        ],
        out_specs=[
            pl.BlockSpec((BR, H), lambda i: (i, 0)),
            pl.BlockSpec((BR, 1), lambda i: (i, 0)),
        ],
        out_shape=[
            jax.ShapeDtypeStruct((NP, H), _f32),
            jax.ShapeDtypeStruct((NP, 1), _f32),
        ],
    )(xp, deg2, W1)


def _tc_mid_body(a_ref, y_ref, dinv_ref, b_ref, w_ref, h_ref, ynext_ref):
    agg = a_ref[0] + a_ref[1] + y_ref[...]
    dinv = dinv_ref[...]
    h = jnp.maximum(agg * dinv + b_ref[...], 0.0)
    h_ref[...] = h
    ynext_ref[...] = jnp.dot(h, w_ref[...], preferred_element_type=_f32) * dinv


def _tc_mid(agg, y_prev, dinv, b, W_next):
    return pl.pallas_call(
        _tc_mid_body,
        grid=(NB,),
        in_specs=[
            pl.BlockSpec((NC, BR, H), lambda i: (0, i, 0)),
            pl.BlockSpec((BR, H), lambda i: (i, 0)),
            pl.BlockSpec((BR, 1), lambda i: (i, 0)),
            pl.BlockSpec((1, H), lambda i: (0, 0)),
            pl.BlockSpec((H, H), lambda i: (0, 0)),
        ],
        out_specs=[
            pl.BlockSpec((BR, H), lambda i: (i, 0)),
            pl.BlockSpec((BR, H), lambda i: (i, 0)),
        ],
        out_shape=[
            jax.ShapeDtypeStruct((NP, H), _f32),
            jax.ShapeDtypeStruct((NP, H), _f32),
        ],
    )(agg, y_prev, dinv, b, W_next)


def _tc_final_body(a_ref, y_ref, dinv_ref, b_ref, h1_ref, h2_ref,
                   wlin_ref, blin_ref, wg_ref, bg_ref, h_ref, gate_ref):
    agg = a_ref[0] + a_ref[1] + y_ref[...]
    h3 = jnp.maximum(agg * dinv_ref[...] + b_ref[...], 0.0)
    hcat = jnp.concatenate([h1_ref[...], h2_ref[...], h3], axis=1)
    h = jnp.maximum(
        jnp.dot(hcat, wlin_ref[...], preferred_element_type=_f32)
        + blin_ref[...], 0.0)
    h_ref[...] = h
    gate_ref[...] = (
        jnp.dot(h, wg_ref[...], preferred_element_type=_f32) + bg_ref[...])


def _tc_final(agg, y3, dinv, b3, h1, h2, Wlin, blin, Wgate, bgate):
    return pl.pallas_call(
        _tc_final_body,
        grid=(NB,),
        in_specs=[
            pl.BlockSpec((NC, BR, H), lambda i: (0, i, 0)),
            pl.BlockSpec((BR, H), lambda i: (i, 0)),
            pl.BlockSpec((BR, 1), lambda i: (i, 0)),
            pl.BlockSpec((1, H), lambda i: (0, 0)),
            pl.BlockSpec((BR, H), lambda i: (i, 0)),
            pl.BlockSpec((BR, H), lambda i: (i, 0)),
            pl.BlockSpec((3 * H, H), lambda i: (0, 0)),
            pl.BlockSpec((1, H), lambda i: (0, 0)),
            pl.BlockSpec((H, 1), lambda i: (0, 0)),
            pl.BlockSpec((1, 1), lambda i: (0, 0)),
        ],
        out_specs=[
            pl.BlockSpec((BR, H), lambda i: (i, 0)),
            pl.BlockSpec((BR, 1), lambda i: (i, 0)),
        ],
        out_shape=[
            jax.ShapeDtypeStruct((NP, H), _f32),
            jax.ShapeDtypeStruct((NP, 1), _f32),
        ],
    )(agg, y3, dinv, b3, h1, h2, Wlin, blin, Wgate, bgate)


def _tc_pool_body(h_ref, gate_ref, seg_ref, out_ref):
    gate = gate_ref[...]
    rowid = lax.broadcasted_iota(_i32, (NP, 1), 0)
    valid = rowid < N
    m = jnp.max(jnp.where(valid, gate, -jnp.inf))
    a = jnp.where(valid, jnp.exp(gate - m), 0.0)
    onehot = jnp.where(
        seg_ref[...] == lax.broadcasted_iota(_i32, (G, NP), 0),
        1.0, 0.0)
    p = jnp.dot(onehot, a * h_ref[...], preferred_element_type=_f32)
    s = jnp.dot(onehot, a, preferred_element_type=_f32)
    out_ref[...] = p / jnp.maximum(s, 1e-16)


def _tc_pool(h, gate, seg):
    return pl.pallas_call(
        _tc_pool_body,
        out_shape=jax.ShapeDtypeStruct((G, H), _f32),
    )(h, gate, seg)


def kernel(batch, x, edge_index, batch_idx, W1, b1, W2, b2, W3, b3,
           Wlin, blin, Wgate, bgate):
    src = edge_index[0].astype(_i32)
    dst = edge_index[1].astype(_i32)
    src_p = jnp.zeros((EP,), _i32).at[:E].set(src)
    dst_p = jnp.full((EP,), DUMP, _i32).at[:E].set(dst)
    xp = jnp.zeros((NP, F_IN), _f32).at[:N].set(x[:, :F_IN])
    seg = jnp.zeros((1, NP), _i32).at[0, :N].set(batch_idx.astype(_i32))

    deg2 = _sc_deg(dst_p).reshape(NC, NP, 1)
    y1, dinv = _tc1(xp, deg2, W1)
    agg1 = _sc_agg(y1, src_p, dst_p)
    h1, y2 = _tc_mid(agg1, y1, dinv, b1.reshape(1, H), W2)
    agg2 = _sc_agg(y2, src_p, dst_p)
    h2, y3 = _tc_mid(agg2, y2, dinv, b2.reshape(1, H), W3)
    agg3 = _sc_agg(y3, src_p, dst_p)
    h, gate = _tc_final(agg3, y3, dinv, b3.reshape(1, H), h1, h2,
                        Wlin, blin.reshape(1, H), Wgate, bgate.reshape(1, 1))
    pooled = _tc_pool(h, gate, seg)
    return h[:N], pooled


# speedup vs baseline: 15.7121x; 15.7121x over previous
import functools

import jax
import jax.numpy as jnp
from jax import lax
from jax.experimental import pallas as pl
from jax.experimental.pallas import tpu as pltpu
from jax.experimental.pallas import tpu_sc as plsc

N = 10000
NP = 10240
F_IN = 128
H = 32
G = 128
E = 320000
NC = 2
NS = 16
NW = NC * NS
CH = 128
EPT = 10112
NCHUNK = EPT // CH
EP = EPT * NW
RPT = NP // NS
DUMP = 10200
NB = 5
BR = NP // NB

_mesh = plsc.VectorSubcoreMesh(core_axis_name="c", subcore_axis_name="s")
_sc_params = pltpu.CompilerParams(use_tc_tiling_on_sc=False)

_f32 = jnp.float32
_i32 = jnp.int32


@functools.partial(
    pl.kernel,
    out_type=jax.ShapeDtypeStruct((NC, NP), _f32),
    mesh=_mesh,
    scratch_types=[
        pltpu.VMEM((CH,), _i32),
        pltpu.VMEM((CH,), _f32),
        pltpu.VMEM((RPT,), _f32),
        pltpu.VMEM_SHARED((NP,), _f32),
    ],
    compiler_params=_sc_params,
)
def _sc_deg(dst_hbm, out_hbm, didx, ones, zbuf, acc):
    c = lax.axis_index("c")
    s = lax.axis_index("s")
    wid = s * NC + c

    def fill(i, _):
        zbuf[pl.ds(i * 16, 16)] = jnp.zeros((16,), _f32)
        return 0

    lax.fori_loop(0, RPT // 16, fill, 0)

    def fill1(i, _):
        ones[pl.ds(i * 16, 16)] = jnp.ones((16,), _f32)
        return 0

    lax.fori_loop(0, CH // 16, fill1, 0)
    pltpu.sync_copy(zbuf, acc.at[pl.ds(s * RPT, RPT)])
    plsc.subcore_barrier()

    def chunk(j, _):
        off = wid * EPT + j * CH
        pltpu.sync_copy(dst_hbm.at[pl.ds(off, CH)], didx)
        pltpu.sync_copy(ones, acc.at[didx], add=True)
        return 0

    lax.fori_loop(0, NCHUNK, chunk, 0)
    plsc.subcore_barrier()
    pltpu.sync_copy(acc.at[pl.ds(s * RPT, RPT)], out_hbm.at[c, pl.ds(s * RPT, RPT)])


@functools.partial(
    pl.kernel,
    out_type=jax.ShapeDtypeStruct((NC, NP, H), _f32),
    mesh=_mesh,
    scratch_types=[
        pltpu.VMEM((CH,), _i32),
        pltpu.VMEM((CH,), _i32),
        pltpu.VMEM((CH, H), _f32),
        pltpu.VMEM((RPT, H), _f32),
        pltpu.VMEM_SHARED((NP, H), _f32),
        pltpu.SemaphoreType.DMA,
    ],
    compiler_params=_sc_params,
)
def _sc_agg(y_hbm, src_hbm, dst_hbm, out_hbm, sidx, didx, rows, zbuf, acc, sem):
    c = lax.axis_index("c")
    s = lax.axis_index("s")
    wid = s * NC + c

    def fill(i, _):
        zbuf[i, pl.ds(0, 16)] = jnp.zeros((16,), _f32)
        zbuf[i, pl.ds(16, 16)] = jnp.zeros((16,), _f32)
        return 0

    lax.fori_loop(0, RPT, fill, 0)
    pltpu.sync_copy(zbuf, acc.at[pl.ds(s * RPT, RPT)])
    plsc.subcore_barrier()

    def chunk(j, _):
        off = wid * EPT + j * CH
        pltpu.sync_copy(src_hbm.at[pl.ds(off, CH)], sidx)
        pltpu.async_copy(y_hbm.at[sidx], rows, sem).wait()
        pltpu.sync_copy(dst_hbm.at[pl.ds(off, CH)], didx)
        pltpu.sync_copy(rows, acc.at[didx], add=True)
        return 0

    lax.fori_loop(0, NCHUNK, chunk, 0)
    plsc.subcore_barrier()
    pltpu.sync_copy(acc.at[pl.ds(s * RPT, RPT)],
                    out_hbm.at[c, pl.ds(s * RPT, RPT)])


def _tc1_body(x_ref, d_ref, w_ref, y_ref, dinv_ref):
    deg = d_ref[0] + d_ref[1] + 1.0
    dinv = lax.rsqrt(deg)
    y = jnp.dot(x_ref[...], w_ref[...], preferred_element_type=_f32)
    y_ref[...] = y * dinv
    dinv_ref[...] = dinv


def _tc1(xp, deg2, W1):
    return pl.pallas_call(
        _tc1_body,
        grid=(NB,),
        in_specs=[
            pl.BlockSpec((BR, F_IN), lambda i: (i, 0)),
            pl.BlockSpec((NC, BR, 1), lambda i: (0, i, 0)),
            pl.BlockSpec((F_IN, H), lambda i: (0, 0)),
        ],
        out_specs=[
            pl.BlockSpec((BR, H), lambda i: (i, 0)),
            pl.BlockSpec((BR, 1), lambda i: (i, 0)),
        ],
        out_shape=[
            jax.ShapeDtypeStruct((NP, H), _f32),
            jax.ShapeDtypeStruct((NP, 1), _f32),
        ],
    )(xp, deg2, W1)


def _tc_mid_body(a_ref, y_ref, dinv_ref, b_ref, w_ref, h_ref, ynext_ref):
    agg = a_ref[0] + a_ref[1] + y_ref[...]
    dinv = dinv_ref[...]
    h = jnp.maximum(agg * dinv + b_ref[...], 0.0)
    h_ref[...] = h
    ynext_ref[...] = jnp.dot(h, w_ref[...], preferred_element_type=_f32) * dinv


def _tc_mid(agg, y_prev, dinv, b, W_next):
    return pl.pallas_call(
        _tc_mid_body,
        grid=(NB,),
        in_specs=[
            pl.BlockSpec((NC, BR, H), lambda i: (0, i, 0)),
            pl.BlockSpec((BR, H), lambda i: (i, 0)),
            pl.BlockSpec((BR, 1), lambda i: (i, 0)),
            pl.BlockSpec((1, H), lambda i: (0, 0)),
            pl.BlockSpec((H, H), lambda i: (0, 0)),
        ],
        out_specs=[
            pl.BlockSpec((BR, H), lambda i: (i, 0)),
            pl.BlockSpec((BR, H), lambda i: (i, 0)),
        ],
        out_shape=[
            jax.ShapeDtypeStruct((NP, H), _f32),
            jax.ShapeDtypeStruct((NP, H), _f32),
        ],
    )(agg, y_prev, dinv, b, W_next)


def _tc_final_body(a_ref, y_ref, dinv_ref, b_ref, h1_ref, h2_ref,
                   wlin_ref, blin_ref, wg_ref, bg_ref, h_ref, gate_ref):
    agg = a_ref[0] + a_ref[1] + y_ref[...]
    h3 = jnp.maximum(agg * dinv_ref[...] + b_ref[...], 0.0)
    hcat = jnp.concatenate([h1_ref[...], h2_ref[...], h3], axis=1)
    h = jnp.maximum(
        jnp.dot(hcat, wlin_ref[...], preferred_element_type=_f32)
        + blin_ref[...], 0.0)
    h_ref[...] = h
    gate_ref[...] = (
        jnp.dot(h, wg_ref[...], preferred_element_type=_f32) + bg_ref[...])


def _tc_final(agg, y3, dinv, b3, h1, h2, Wlin, blin, Wgate, bgate):
    return pl.pallas_call(
        _tc_final_body,
        grid=(NB,),
        in_specs=[
            pl.BlockSpec((NC, BR, H), lambda i: (0, i, 0)),
            pl.BlockSpec((BR, H), lambda i: (i, 0)),
            pl.BlockSpec((BR, 1), lambda i: (i, 0)),
            pl.BlockSpec((1, H), lambda i: (0, 0)),
            pl.BlockSpec((BR, H), lambda i: (i, 0)),
            pl.BlockSpec((BR, H), lambda i: (i, 0)),
            pl.BlockSpec((3 * H, H), lambda i: (0, 0)),
            pl.BlockSpec((1, H), lambda i: (0, 0)),
            pl.BlockSpec((H, 1), lambda i: (0, 0)),
            pl.BlockSpec((1, 1), lambda i: (0, 0)),
        ],
        out_specs=[
            pl.BlockSpec((BR, H), lambda i: (i, 0)),
            pl.BlockSpec((BR, 1), lambda i: (i, 0)),
        ],
        out_shape=[
            jax.ShapeDtypeStruct((NP, H), _f32),
            jax.ShapeDtypeStruct((NP, 1), _f32),
        ],
    )(agg, y3, dinv, b3, h1, h2, Wlin, blin, Wgate, bgate)


def _tc_pool_body(h_ref, gate_ref, seg_ref, out_ref):
    gate = gate_ref[...]
    rowid = lax.broadcasted_iota(_i32, (NP, 1), 0)
    valid = rowid < N
    m = jnp.max(jnp.where(valid, gate, -jnp.inf))
    a = jnp.where(valid, jnp.exp(gate - m), 0.0)
    onehot = jnp.where(
        seg_ref[...] == lax.broadcasted_iota(_i32, (G, NP), 0),
        1.0, 0.0)
    p = jnp.dot(onehot, a * h_ref[...], preferred_element_type=_f32)
    s = jnp.dot(onehot, a, preferred_element_type=_f32)
    out_ref[...] = p / jnp.maximum(s, 1e-16)


def _tc_pool(h, gate, seg):
    return pl.pallas_call(
        _tc_pool_body,
        out_shape=jax.ShapeDtypeStruct((G, H), _f32),
    )(h, gate, seg)


def kernel(batch, x, edge_index, batch_idx, W1, b1, W2, b2, W3, b3,
           Wlin, blin, Wgate, bgate):
    src = edge_index[0].astype(_i32)
    dst = edge_index[1].astype(_i32)
    src_p = jnp.zeros((EP,), _i32).at[:E].set(src)
    dst_p = jnp.full((EP,), DUMP, _i32).at[:E].set(dst)
    xp = jnp.zeros((NP, F_IN), _f32).at[:N].set(x[:, :F_IN])
    seg = jnp.zeros((1, NP), _i32).at[0, :N].set(batch_idx.astype(_i32))

    deg2 = _sc_deg(dst_p).reshape(NC, NP, 1)
    y1, dinv = _tc1(xp, deg2, W1)
    agg1 = _sc_agg(y1, src_p, dst_p)
    h1, y2 = _tc_mid(agg1, y1, dinv, b1.reshape(1, H), W2)
    agg2 = _sc_agg(y2, src_p, dst_p)
    h2, y3 = _tc_mid(agg2, y2, dinv, b2.reshape(1, H), W3)
    agg3 = _sc_agg(y3, src_p, dst_p)
    h, gate = _tc_final(agg3, y3, dinv, b3.reshape(1, H), h1, h2,
                        Wlin, blin.reshape(1, H), Wgate, bgate.reshape(1, 1))
    pooled = _tc_pool(h, gate, seg)
    return h[:N], pooled


# speedup vs baseline: 22.6761x; 1.4432x over previous
import functools

import jax
import jax.numpy as jnp
from jax import lax
from jax.experimental import pallas as pl
from jax.experimental.pallas import tpu as pltpu
from jax.experimental.pallas import tpu_sc as plsc

N = 10000
NP = 10240
F_IN = 128
H = 32
G = 128
E = 320000
NC = 2
NS = 16
NW = NC * NS
CH = 128
EPT = 10240
NCHUNK = EPT // CH
EP = EPT * NW
NBUF = 8
LA = 4
RPT = NP // NS
DUMP = 10200
NB = 5
BR = NP // NB

_mesh = plsc.VectorSubcoreMesh(core_axis_name="c", subcore_axis_name="s")
_sc_params = pltpu.CompilerParams(use_tc_tiling_on_sc=False)

_f32 = jnp.float32
_i32 = jnp.int32


@functools.partial(
    pl.kernel,
    out_type=jax.ShapeDtypeStruct((NC, NP), _f32),
    mesh=_mesh,
    scratch_types=[
        pltpu.VMEM((NCHUNK, CH), _i32),
        pltpu.VMEM((CH,), _f32),
        pltpu.VMEM((RPT,), _f32),
        pltpu.VMEM_SHARED((NP,), _f32),
        pltpu.SemaphoreType.DMA,
        pltpu.SemaphoreType.DMA,
        pltpu.SemaphoreType.DMA,
        pltpu.SemaphoreType.DMA,
    ],
    compiler_params=_sc_params,
)
def _sc_deg(dst_hbm, out_hbm, didx, ones, zbuf, acc, s0, s1, s2, s3):
    c = lax.axis_index("c")
    s = lax.axis_index("s")
    wid = s * NC + c
    sems = [s0, s1, s2, s3]

    pltpu.sync_copy(dst_hbm.at[wid], didx)

    def fill(i, _):
        zbuf[pl.ds(i * 16, 16)] = jnp.zeros((16,), _f32)
        return 0

    lax.fori_loop(0, RPT // 16, fill, 0)

    def fill1(i, _):
        ones[pl.ds(i * 16, 16)] = jnp.ones((16,), _f32)
        return 0

    lax.fori_loop(0, CH // 16, fill1, 0)
    pltpu.sync_copy(zbuf, acc.at[pl.ds(s * RPT, RPT)])
    plsc.subcore_barrier()

    def outer(jj, _):
        for b in range(4):
            j = jj * 4 + b

            @pl.when(j >= 4)
            def _wait():
                pltpu.make_async_copy(ones, acc.at[didx.at[j - 4]],
                                      sems[b]).wait()

            pltpu.async_copy(ones, acc.at[didx.at[j]], sems[b], add=True)
        return 0

    lax.fori_loop(0, NCHUNK // 4, outer, 0)
    for b in range(4):
        pltpu.make_async_copy(ones, acc.at[didx.at[NCHUNK - 4 + b]],
                              sems[b]).wait()
    plsc.subcore_barrier()
    pltpu.sync_copy(acc.at[pl.ds(s * RPT, RPT)], out_hbm.at[c, pl.ds(s * RPT, RPT)])


@functools.partial(
    pl.kernel,
    out_type=jax.ShapeDtypeStruct((NC, NP, H), _f32),
    mesh=_mesh,
    scratch_types=(
        [pltpu.VMEM((NCHUNK, CH), _i32)] * 2
        + [pltpu.VMEM((CH, H), _f32)] * NBUF
        + [pltpu.VMEM((RPT, H), _f32),
           pltpu.VMEM_SHARED((NP, H), _f32)]
        + [pltpu.SemaphoreType.DMA] * (2 * NBUF)
    ),
    compiler_params=_sc_params,
)
def _sc_agg(y_hbm, src_hbm, dst_hbm, out_hbm, sidx, didx, *rest):
    rows = rest[:NBUF]
    zbuf, acc = rest[NBUF], rest[NBUF + 1]
    semg = rest[NBUF + 2:2 * NBUF + 2]
    sems = rest[2 * NBUF + 2:]
    c = lax.axis_index("c")
    s = lax.axis_index("s")
    wid = s * NC + c

    pltpu.sync_copy(src_hbm.at[wid], sidx)
    pltpu.sync_copy(dst_hbm.at[wid], didx)
    for b in range(LA):
        pltpu.async_copy(y_hbm.at[sidx.at[b]], rows[b], semg[b])

    def fill(i, _):
        zbuf[i, pl.ds(0, 16)] = jnp.zeros((16,), _f32)
        zbuf[i, pl.ds(16, 16)] = jnp.zeros((16,), _f32)
        return 0

    lax.fori_loop(0, RPT, fill, 0)
    pltpu.sync_copy(zbuf, acc.at[pl.ds(s * RPT, RPT)])
    plsc.subcore_barrier()

    def outer(jj, _):
        for b in range(NBUF):
            j = jj * NBUF + b
            bf = (b + LA) % NBUF

            @pl.when(j >= LA)
            def _drain():
                pltpu.make_async_copy(rows[bf], acc.at[didx.at[j - LA]],
                                      sems[bf]).wait()

            @pl.when(j + LA < NCHUNK)
            def _prefetch():
                pltpu.async_copy(y_hbm.at[sidx.at[j + LA]], rows[bf], semg[bf])

            pltpu.make_async_copy(y_hbm.at[sidx.at[j]], rows[b], semg[b]).wait()
            pltpu.async_copy(rows[b], acc.at[didx.at[j]], sems[b], add=True)
        return 0

    lax.fori_loop(0, NCHUNK // NBUF, outer, 0)
    for b in range(LA):
        j = NCHUNK - LA + b
        pltpu.make_async_copy(rows[j % NBUF], acc.at[didx.at[j]],
                              sems[j % NBUF]).wait()
    plsc.subcore_barrier()
    pltpu.sync_copy(acc.at[pl.ds(s * RPT, RPT)],
                    out_hbm.at[c, pl.ds(s * RPT, RPT)])


def _tc1_body(x_ref, d_ref, w_ref, y_ref, dinv_ref):
    deg = d_ref[0] + d_ref[1] + 1.0
    dinv = lax.rsqrt(deg)
    y = jnp.dot(x_ref[...], w_ref[...], preferred_element_type=_f32)
    y_ref[...] = y * dinv
    dinv_ref[...] = dinv


def _tc1(xp, deg2, W1):
    return pl.pallas_call(
        _tc1_body,
        grid=(NB,),
        in_specs=[
            pl.BlockSpec((BR, F_IN), lambda i: (i, 0)),
            pl.BlockSpec((NC, BR, 1), lambda i: (0, i, 0)),
            pl.BlockSpec((F_IN, H), lambda i: (0, 0)),
        ],
        out_specs=[
            pl.BlockSpec((BR, H), lambda i: (i, 0)),
            pl.BlockSpec((BR, 1), lambda i: (i, 0)),
        ],
        out_shape=[
            jax.ShapeDtypeStruct((NP, H), _f32),
            jax.ShapeDtypeStruct((NP, 1), _f32),
        ],
    )(xp, deg2, W1)


def _tc_mid_body(a_ref, y_ref, dinv_ref, b_ref, w_ref, h_ref, ynext_ref):
    agg = a_ref[0] + a_ref[1] + y_ref[...]
    dinv = dinv_ref[...]
    h = jnp.maximum(agg * dinv + b_ref[...], 0.0)
    h_ref[...] = h
    ynext_ref[...] = jnp.dot(h, w_ref[...], preferred_element_type=_f32) * dinv


def _tc_mid(agg, y_prev, dinv, b, W_next):
    return pl.pallas_call(
        _tc_mid_body,
        grid=(NB,),
        in_specs=[
            pl.BlockSpec((NC, BR, H), lambda i: (0, i, 0)),
            pl.BlockSpec((BR, H), lambda i: (i, 0)),
            pl.BlockSpec((BR, 1), lambda i: (i, 0)),
            pl.BlockSpec((1, H), lambda i: (0, 0)),
            pl.BlockSpec((H, H), lambda i: (0, 0)),
        ],
        out_specs=[
            pl.BlockSpec((BR, H), lambda i: (i, 0)),
            pl.BlockSpec((BR, H), lambda i: (i, 0)),
        ],
        out_shape=[
            jax.ShapeDtypeStruct((NP, H), _f32),
            jax.ShapeDtypeStruct((NP, H), _f32),
        ],
    )(agg, y_prev, dinv, b, W_next)


def _tc_final_body(a_ref, y_ref, dinv_ref, b_ref, h1_ref, h2_ref,
                   wlin_ref, blin_ref, wg_ref, bg_ref, h_ref, gate_ref):
    agg = a_ref[0] + a_ref[1] + y_ref[...]
    h3 = jnp.maximum(agg * dinv_ref[...] + b_ref[...], 0.0)
    hcat = jnp.concatenate([h1_ref[...], h2_ref[...], h3], axis=1)
    h = jnp.maximum(
        jnp.dot(hcat, wlin_ref[...], preferred_element_type=_f32)
        + blin_ref[...], 0.0)
    h_ref[...] = h
    gate_ref[...] = (
        jnp.dot(h, wg_ref[...], preferred_element_type=_f32) + bg_ref[...])


def _tc_final(agg, y3, dinv, b3, h1, h2, Wlin, blin, Wgate, bgate):
    return pl.pallas_call(
        _tc_final_body,
        grid=(NB,),
        in_specs=[
            pl.BlockSpec((NC, BR, H), lambda i: (0, i, 0)),
            pl.BlockSpec((BR, H), lambda i: (i, 0)),
            pl.BlockSpec((BR, 1), lambda i: (i, 0)),
            pl.BlockSpec((1, H), lambda i: (0, 0)),
            pl.BlockSpec((BR, H), lambda i: (i, 0)),
            pl.BlockSpec((BR, H), lambda i: (i, 0)),
            pl.BlockSpec((3 * H, H), lambda i: (0, 0)),
            pl.BlockSpec((1, H), lambda i: (0, 0)),
            pl.BlockSpec((H, 1), lambda i: (0, 0)),
            pl.BlockSpec((1, 1), lambda i: (0, 0)),
        ],
        out_specs=[
            pl.BlockSpec((BR, H), lambda i: (i, 0)),
            pl.BlockSpec((BR, 1), lambda i: (i, 0)),
        ],
        out_shape=[
            jax.ShapeDtypeStruct((NP, H), _f32),
            jax.ShapeDtypeStruct((NP, 1), _f32),
        ],
    )(agg, y3, dinv, b3, h1, h2, Wlin, blin, Wgate, bgate)


def _tc_pool_body(h_ref, gate_ref, seg_ref, out_ref):
    gate = gate_ref[...]
    rowid = lax.broadcasted_iota(_i32, (NP, 1), 0)
    valid = rowid < N
    m = jnp.max(jnp.where(valid, gate, -jnp.inf))
    a = jnp.where(valid, jnp.exp(gate - m), 0.0)
    onehot = jnp.where(
        seg_ref[...] == lax.broadcasted_iota(_i32, (G, NP), 0),
        1.0, 0.0)
    p = jnp.dot(onehot, a * h_ref[...], preferred_element_type=_f32)
    s = jnp.dot(onehot, a, preferred_element_type=_f32)
    out_ref[...] = p / jnp.maximum(s, 1e-16)


def _tc_pool(h, gate, seg):
    return pl.pallas_call(
        _tc_pool_body,
        out_shape=jax.ShapeDtypeStruct((G, H), _f32),
    )(h, gate, seg)


def kernel(batch, x, edge_index, batch_idx, W1, b1, W2, b2, W3, b3,
           Wlin, blin, Wgate, bgate):
    src = edge_index[0].astype(_i32)
    dst = edge_index[1].astype(_i32)
    src_p = jnp.zeros((EP,), _i32).at[:E].set(src).reshape(NW, NCHUNK, CH)
    dst_p = jnp.full((EP,), DUMP, _i32).at[:E].set(dst).reshape(NW, NCHUNK, CH)
    xp = jnp.zeros((NP, F_IN), _f32).at[:N].set(x[:, :F_IN])
    seg = jnp.zeros((1, NP), _i32).at[0, :N].set(batch_idx.astype(_i32))

    deg2 = _sc_deg(dst_p).reshape(NC, NP, 1)
    y1, dinv = _tc1(xp, deg2, W1)
    agg1 = _sc_agg(y1, src_p, dst_p)
    h1, y2 = _tc_mid(agg1, y1, dinv, b1.reshape(1, H), W2)
    agg2 = _sc_agg(y2, src_p, dst_p)
    h2, y3 = _tc_mid(agg2, y2, dinv, b2.reshape(1, H), W3)
    agg3 = _sc_agg(y3, src_p, dst_p)
    h, gate = _tc_final(agg3, y3, dinv, b3.reshape(1, H), h1, h2,
                        Wlin, blin.reshape(1, H), Wgate, bgate.reshape(1, 1))
    pooled = _tc_pool(h, gate, seg)
    return h[:N], pooled


# speedup vs baseline: 22.8415x; 1.0073x over previous
import functools

import jax
import jax.numpy as jnp
from jax import lax
from jax.experimental import pallas as pl
from jax.experimental.pallas import tpu as pltpu
from jax.experimental.pallas import tpu_sc as plsc

N = 10000
NP = 10240
F_IN = 128
H = 32
G = 128
E = 320000
NC = 2
NS = 16
NW = NC * NS
CH = 256
EPT = 10240
NCHUNK = EPT // CH
EP = EPT * NW
NBUF = 8
LA = 4
RPT = NP // NS
DUMP = 10200
NB = 5
BR = NP // NB

_mesh = plsc.VectorSubcoreMesh(core_axis_name="c", subcore_axis_name="s")
_sc_params = pltpu.CompilerParams(use_tc_tiling_on_sc=False)

_f32 = jnp.float32
_i32 = jnp.int32


@functools.partial(
    pl.kernel,
    out_type=jax.ShapeDtypeStruct((NC, NP), _f32),
    mesh=_mesh,
    scratch_types=[
        pltpu.VMEM((NCHUNK, CH), _i32),
        pltpu.VMEM((CH,), _f32),
        pltpu.VMEM((RPT,), _f32),
        pltpu.VMEM_SHARED((NP,), _f32),
    ] + [pltpu.SemaphoreType.DMA] * 4,
    compiler_params=_sc_params,
)
def _sc_deg(dst_hbm, out_hbm, didx, ones, zbuf, acc, *sems):
    c = lax.axis_index("c")
    s = lax.axis_index("s")
    wid = s * NC + c

    pltpu.sync_copy(dst_hbm.at[wid], didx)

    def fill(i, _):
        zbuf[pl.ds(i * 16, 16)] = jnp.zeros((16,), _f32)
        return 0

    lax.fori_loop(0, RPT // 16, fill, 0)

    def fill1(i, _):
        ones[pl.ds(i * 16, 16)] = jnp.ones((16,), _f32)
        return 0

    lax.fori_loop(0, CH // 16, fill1, 0)
    pltpu.sync_copy(zbuf, acc.at[pl.ds(s * RPT, RPT)])
    plsc.subcore_barrier()

    def outer(jj, _):
        for b in range(4):
            j = jj * 4 + b

            @pl.when(j >= 4)
            def _wait():
                pltpu.make_async_copy(ones, acc.at[didx.at[j - 4]],
                                      sems[b]).wait()

            pltpu.async_copy(ones, acc.at[didx.at[j]], sems[b], add=True)
        return 0

    lax.fori_loop(0, NCHUNK // 4, outer, 0)
    for b in range(4):
        pltpu.make_async_copy(ones, acc.at[didx.at[NCHUNK - 4 + b]],
                              sems[b]).wait()
    plsc.subcore_barrier()
    pltpu.sync_copy(acc.at[pl.ds(s * RPT, RPT)], out_hbm.at[c, pl.ds(s * RPT, RPT)])


@functools.partial(
    pl.kernel,
    out_type=jax.ShapeDtypeStruct((NC, NP, H), _f32),
    mesh=_mesh,
    scratch_types=(
        [pltpu.VMEM((NCHUNK, CH), _i32)] * 2
        + [pltpu.VMEM((CH, H), _f32)] * NBUF
        + [pltpu.VMEM((RPT, H), _f32),
           pltpu.VMEM_SHARED((NP, H), _f32)]
        + [pltpu.SemaphoreType.DMA] * (2 * NBUF)
    ),
    compiler_params=_sc_params,
)
def _sc_agg(y_hbm, src_hbm, dst_hbm, out_hbm, sidx, didx, *rest):
    rows = rest[:NBUF]
    zbuf, acc = rest[NBUF], rest[NBUF + 1]
    semg = rest[NBUF + 2:2 * NBUF + 2]
    sems = rest[2 * NBUF + 2:]
    c = lax.axis_index("c")
    s = lax.axis_index("s")
    wid = s * NC + c

    pltpu.sync_copy(src_hbm.at[wid], sidx)
    pltpu.sync_copy(dst_hbm.at[wid], didx)
    for b in range(LA):
        pltpu.async_copy(y_hbm.at[sidx.at[b]], rows[b], semg[b])

    def fill(i, _):
        zbuf[i, pl.ds(0, 16)] = jnp.zeros((16,), _f32)
        zbuf[i, pl.ds(16, 16)] = jnp.zeros((16,), _f32)
        return 0

    lax.fori_loop(0, RPT, fill, 0)
    pltpu.sync_copy(zbuf, acc.at[pl.ds(s * RPT, RPT)])
    plsc.subcore_barrier()

    def outer(jj, _):
        for b in range(NBUF):
            j = jj * NBUF + b
            bf = (b + LA) % NBUF

            @pl.when(j >= LA)
            def _drain():
                pltpu.make_async_copy(rows[bf], acc.at[didx.at[j - LA]],
                                      sems[bf]).wait()

            @pl.when(j + LA < NCHUNK)
            def _prefetch():
                pltpu.async_copy(y_hbm.at[sidx.at[j + LA]], rows[bf], semg[bf])

            pltpu.make_async_copy(y_hbm.at[sidx.at[j]], rows[b], semg[b]).wait()
            pltpu.async_copy(rows[b], acc.at[didx.at[j]], sems[b], add=True)
        return 0

    lax.fori_loop(0, NCHUNK // NBUF, outer, 0)
    for b in range(LA):
        j = NCHUNK - LA + b
        pltpu.make_async_copy(rows[j % NBUF], acc.at[didx.at[j]],
                              sems[j % NBUF]).wait()
    plsc.subcore_barrier()
    pltpu.sync_copy(acc.at[pl.ds(s * RPT, RPT)],
                    out_hbm.at[c, pl.ds(s * RPT, RPT)])


def _tc1_body(x_ref, d_ref, w_ref, y_ref, dinv_ref):
    deg = d_ref[0] + d_ref[1] + 1.0
    dinv = lax.rsqrt(deg)
    y = jnp.dot(x_ref[...], w_ref[...], preferred_element_type=_f32)
    y_ref[...] = y * dinv
    dinv_ref[...] = dinv


def _tc1(xp, deg2, W1):
    return pl.pallas_call(
        _tc1_body,
        grid=(NB,),
        in_specs=[
            pl.BlockSpec((BR, F_IN), lambda i: (i, 0)),
            pl.BlockSpec((NC, BR, 1), lambda i: (0, i, 0)),
            pl.BlockSpec((F_IN, H), lambda i: (0, 0)),
        ],
        out_specs=[
            pl.BlockSpec((BR, H), lambda i: (i, 0)),
            pl.BlockSpec((BR, 1), lambda i: (i, 0)),
        ],
        out_shape=[
            jax.ShapeDtypeStruct((NP, H), _f32),
            jax.ShapeDtypeStruct((NP, 1), _f32),
        ],
    )(xp, deg2, W1)


def _tc_mid_body(a_ref, y_ref, dinv_ref, b_ref, w_ref, h_ref, ynext_ref):
    agg = a_ref[0] + a_ref[1] + y_ref[...]
    dinv = dinv_ref[...]
    h = jnp.maximum(agg * dinv + b_ref[...], 0.0)
    h_ref[...] = h
    ynext_ref[...] = jnp.dot(h, w_ref[...], preferred_element_type=_f32) * dinv


def _tc_mid(agg, y_prev, dinv, b, W_next):
    return pl.pallas_call(
        _tc_mid_body,
        grid=(NB,),
        in_specs=[
            pl.BlockSpec((NC, BR, H), lambda i: (0, i, 0)),
            pl.BlockSpec((BR, H), lambda i: (i, 0)),
            pl.BlockSpec((BR, 1), lambda i: (i, 0)),
            pl.BlockSpec((1, H), lambda i: (0, 0)),
            pl.BlockSpec((H, H), lambda i: (0, 0)),
        ],
        out_specs=[
            pl.BlockSpec((BR, H), lambda i: (i, 0)),
            pl.BlockSpec((BR, H), lambda i: (i, 0)),
        ],
        out_shape=[
            jax.ShapeDtypeStruct((NP, H), _f32),
            jax.ShapeDtypeStruct((NP, H), _f32),
        ],
    )(agg, y_prev, dinv, b, W_next)


def _tc_final_body(a_ref, y_ref, dinv_ref, b_ref, h1_ref, h2_ref,
                   wlin_ref, blin_ref, wg_ref, bg_ref, h_ref, gate_ref):
    agg = a_ref[0] + a_ref[1] + y_ref[...]
    h3 = jnp.maximum(agg * dinv_ref[...] + b_ref[...], 0.0)
    hcat = jnp.concatenate([h1_ref[...], h2_ref[...], h3], axis=1)
    h = jnp.maximum(
        jnp.dot(hcat, wlin_ref[...], preferred_element_type=_f32)
        + blin_ref[...], 0.0)
    h_ref[...] = h
    gate_ref[...] = (
        jnp.dot(h, wg_ref[...], preferred_element_type=_f32) + bg_ref[...])


def _tc_final(agg, y3, dinv, b3, h1, h2, Wlin, blin, Wgate, bgate):
    return pl.pallas_call(
        _tc_final_body,
        grid=(NB,),
        in_specs=[
            pl.BlockSpec((NC, BR, H), lambda i: (0, i, 0)),
            pl.BlockSpec((BR, H), lambda i: (i, 0)),
            pl.BlockSpec((BR, 1), lambda i: (i, 0)),
            pl.BlockSpec((1, H), lambda i: (0, 0)),
            pl.BlockSpec((BR, H), lambda i: (i, 0)),
            pl.BlockSpec((BR, H), lambda i: (i, 0)),
            pl.BlockSpec((3 * H, H), lambda i: (0, 0)),
            pl.BlockSpec((1, H), lambda i: (0, 0)),
            pl.BlockSpec((H, 1), lambda i: (0, 0)),
            pl.BlockSpec((1, 1), lambda i: (0, 0)),
        ],
        out_specs=[
            pl.BlockSpec((BR, H), lambda i: (i, 0)),
            pl.BlockSpec((BR, 1), lambda i: (i, 0)),
        ],
        out_shape=[
            jax.ShapeDtypeStruct((NP, H), _f32),
            jax.ShapeDtypeStruct((NP, 1), _f32),
        ],
    )(agg, y3, dinv, b3, h1, h2, Wlin, blin, Wgate, bgate)


def _tc_pool_body(h_ref, gate_ref, seg_ref, out_ref):
    gate = gate_ref[...]
    rowid = lax.broadcasted_iota(_i32, (NP, 1), 0)
    valid = rowid < N
    m = jnp.max(jnp.where(valid, gate, -jnp.inf))
    a = jnp.where(valid, jnp.exp(gate - m), 0.0)
    onehot = jnp.where(
        seg_ref[...] == lax.broadcasted_iota(_i32, (G, NP), 0),
        1.0, 0.0)
    p = jnp.dot(onehot, a * h_ref[...], preferred_element_type=_f32)
    s = jnp.dot(onehot, a, preferred_element_type=_f32)
    out_ref[...] = p / jnp.maximum(s, 1e-16)


def _tc_pool(h, gate, seg):
    return pl.pallas_call(
        _tc_pool_body,
        out_shape=jax.ShapeDtypeStruct((G, H), _f32),
    )(h, gate, seg)


def kernel(batch, x, edge_index, batch_idx, W1, b1, W2, b2, W3, b3,
           Wlin, blin, Wgate, bgate):
    src = edge_index[0].astype(_i32)
    dst = edge_index[1].astype(_i32)
    src_p = jnp.zeros((EP,), _i32).at[:E].set(src).reshape(NW, NCHUNK, CH)
    dst_p = jnp.full((EP,), DUMP, _i32).at[:E].set(dst).reshape(NW, NCHUNK, CH)
    xp = jnp.zeros((NP, F_IN), _f32).at[:N].set(x[:, :F_IN])
    seg = jnp.zeros((1, NP), _i32).at[0, :N].set(batch_idx.astype(_i32))

    deg2 = _sc_deg(dst_p).reshape(NC, NP, 1)
    y1, dinv = _tc1(xp, deg2, W1)
    agg1 = _sc_agg(y1, src_p, dst_p)
    h1, y2 = _tc_mid(agg1, y1, dinv, b1.reshape(1, H), W2)
    agg2 = _sc_agg(y2, src_p, dst_p)
    h2, y3 = _tc_mid(agg2, y2, dinv, b2.reshape(1, H), W3)
    agg3 = _sc_agg(y3, src_p, dst_p)
    h, gate = _tc_final(agg3, y3, dinv, b3.reshape(1, H), h1, h2,
                        Wlin, blin.reshape(1, H), Wgate, bgate.reshape(1, 1))
    pooled = _tc_pool(h, gate, seg)
    return h[:N], pooled


# speedup vs baseline: 22.8658x; 1.0011x over previous
import functools

import jax
import jax.numpy as jnp
from jax import lax
from jax.experimental import pallas as pl
from jax.experimental.pallas import tpu as pltpu
from jax.experimental.pallas import tpu_sc as plsc

N = 10000
NP = 10240
F_IN = 128
H = 32
G = 128
E = 320000
NC = 2
NS = 16
NW = NC * NS
CH = 256
EP = 327680
NCHT = EP // CH
NCH0 = 64
NCH1 = 16
NBUF = 8
LA = 4
RPT = NP // NS
DUMP = 10200
NB = 5
BR = NP // NB

_mesh = plsc.VectorSubcoreMesh(core_axis_name="c", subcore_axis_name="s")
_sc_params = pltpu.CompilerParams(use_tc_tiling_on_sc=False)

_f32 = jnp.float32
_i32 = jnp.int32


@functools.partial(
    pl.kernel,
    out_type=jax.ShapeDtypeStruct((NC, NP), _f32),
    mesh=_mesh,
    scratch_types=[
        pltpu.VMEM((NCH0, CH), _i32),
        pltpu.VMEM((CH,), _f32),
        pltpu.VMEM((RPT,), _f32),
        pltpu.VMEM_SHARED((NP,), _f32),
    ] + [pltpu.SemaphoreType.DMA] * 4,
    compiler_params=_sc_params,
)
def _sc_deg(ei_hbm, out_hbm, didx, ones, zbuf, acc, *sems):
    c = lax.axis_index("c")
    s = lax.axis_index("s")

    def fill(i, _):
        zbuf[pl.ds(i * 16, 16)] = jnp.zeros((16,), _f32)
        return 0

    lax.fori_loop(0, RPT // 16, fill, 0)

    def fill1(i, _):
        ones[pl.ds(i * 16, 16)] = jnp.ones((16,), _f32)
        return 0

    lax.fori_loop(0, CH // 16, fill1, 0)

    def pipe(nch, base):
        pltpu.sync_copy(ei_hbm.at[1, pl.ds(base, nch)], didx.at[pl.ds(0, nch)])
        pltpu.sync_copy(zbuf, acc.at[pl.ds(s * RPT, RPT)])
        plsc.subcore_barrier()

        def outer(jj, _):
            for b in range(4):
                j = jj * 4 + b

                @pl.when(j >= 4)
                def _wait():
                    pltpu.make_async_copy(ones, acc.at[didx.at[j - 4]],
                                          sems[b]).wait()

                pltpu.async_copy(ones, acc.at[didx.at[j]], sems[b], add=True)
            return 0

        lax.fori_loop(0, nch // 4, outer, 0)
        for b in range(4):
            pltpu.make_async_copy(ones, acc.at[didx.at[nch - 4 + b]],
                                  sems[b]).wait()

    @pl.when(c == 0)
    def _sc0():
        pipe(NCH0, s * NCH0)

    @pl.when(c == 1)
    def _sc1():
        pipe(NCH1, NS * NCH0 + s * NCH1)

    plsc.subcore_barrier()
    pltpu.sync_copy(acc.at[pl.ds(s * RPT, RPT)], out_hbm.at[c, pl.ds(s * RPT, RPT)])


@functools.partial(
    pl.kernel,
    out_type=jax.ShapeDtypeStruct((NC, NP, H), _f32),
    mesh=_mesh,
    scratch_types=(
        [pltpu.VMEM((NCH0, CH), _i32)] * 2
        + [pltpu.VMEM((CH, H), _f32)] * NBUF
        + [pltpu.VMEM((RPT // 10, H), _f32),
           pltpu.VMEM_SHARED((NP, H), _f32)]
        + [pltpu.SemaphoreType.DMA] * (2 * NBUF)
    ),
    compiler_params=_sc_params,
)
def _sc_agg(y_hbm, ei_hbm, out_hbm, sidx, didx, *rest):
    rows = rest[:NBUF]
    zbuf, acc = rest[NBUF], rest[NBUF + 1]
    semg = rest[NBUF + 2:2 * NBUF + 2]
    sems = rest[2 * NBUF + 2:]
    c = lax.axis_index("c")
    s = lax.axis_index("s")

    def fill(i, _):
        zbuf[i, pl.ds(0, 16)] = jnp.zeros((16,), _f32)
        zbuf[i, pl.ds(16, 16)] = jnp.zeros((16,), _f32)
        return 0

    lax.fori_loop(0, RPT // 10, fill, 0)

    def pipe(nch, base):
        pltpu.sync_copy(ei_hbm.at[0, pl.ds(base, nch)], sidx.at[pl.ds(0, nch)])
        pltpu.sync_copy(ei_hbm.at[1, pl.ds(base, nch)], didx.at[pl.ds(0, nch)])
        for b in range(LA):
            pltpu.async_copy(y_hbm.at[sidx.at[b]], rows[b], semg[b])

        def zcopy(k, _):
            pltpu.sync_copy(zbuf,
                            acc.at[pl.ds(s * RPT + k * (RPT // 10), RPT // 10)])
            return 0

        lax.fori_loop(0, 10, zcopy, 0)
        plsc.subcore_barrier()

        def outer(jj, _):
            for b in range(NBUF):
                j = jj * NBUF + b
                bf = (b + LA) % NBUF

                @pl.when(j >= LA)
                def _drain():
                    pltpu.make_async_copy(rows[bf], acc.at[didx.at[j - LA]],
                                          sems[bf]).wait()

                @pl.when(j + LA < nch)
                def _prefetch():
                    pltpu.async_copy(y_hbm.at[sidx.at[j + LA]], rows[bf],
                                     semg[bf])

                pltpu.make_async_copy(y_hbm.at[sidx.at[j]], rows[b],
                                      semg[b]).wait()
                pltpu.async_copy(rows[b], acc.at[didx.at[j]], sems[b], add=True)
            return 0

        lax.fori_loop(0, nch // NBUF, outer, 0)
        for b in range(LA):
            j = nch - LA + b
            pltpu.make_async_copy(rows[j % NBUF], acc.at[didx.at[j]],
                                  sems[j % NBUF]).wait()

    @pl.when(c == 0)
    def _sc0():
        pipe(NCH0, s * NCH0)

    @pl.when(c == 1)
    def _sc1():
        pipe(NCH1, NS * NCH0 + s * NCH1)

    plsc.subcore_barrier()
    pltpu.sync_copy(acc.at[pl.ds(s * RPT, RPT)],
                    out_hbm.at[c, pl.ds(s * RPT, RPT)])


def _tc1_body(x_ref, d_ref, w_ref, y_ref, dinv_ref):
    deg = d_ref[0] + d_ref[1] + 1.0
    dinv = lax.rsqrt(deg)
    y = jnp.dot(x_ref[...], w_ref[...], preferred_element_type=_f32)
    y_ref[...] = y * dinv
    dinv_ref[...] = dinv


def _tc1(xp, deg2, W1):
    return pl.pallas_call(
        _tc1_body,
        grid=(NB,),
        in_specs=[
            pl.BlockSpec((BR, F_IN), lambda i: (i, 0)),
            pl.BlockSpec((NC, BR, 1), lambda i: (0, i, 0)),
            pl.BlockSpec((F_IN, H), lambda i: (0, 0)),
        ],
        out_specs=[
            pl.BlockSpec((BR, H), lambda i: (i, 0)),
            pl.BlockSpec((BR, 1), lambda i: (i, 0)),
        ],
        out_shape=[
            jax.ShapeDtypeStruct((NP, H), _f32),
            jax.ShapeDtypeStruct((NP, 1), _f32),
        ],
    )(xp, deg2, W1)


def _tc_mid_body(a_ref, y_ref, dinv_ref, b_ref, w_ref, h_ref, ynext_ref):
    agg = a_ref[0] + a_ref[1] + y_ref[...]
    dinv = dinv_ref[...]
    h = jnp.maximum(agg * dinv + b_ref[...], 0.0)
    h_ref[...] = h
    ynext_ref[...] = jnp.dot(h, w_ref[...], preferred_element_type=_f32) * dinv


def _tc_mid(agg, y_prev, dinv, b, W_next):
    return pl.pallas_call(
        _tc_mid_body,
        grid=(NB,),
        in_specs=[
            pl.BlockSpec((NC, BR, H), lambda i: (0, i, 0)),
            pl.BlockSpec((BR, H), lambda i: (i, 0)),
            pl.BlockSpec((BR, 1), lambda i: (i, 0)),
            pl.BlockSpec((1, H), lambda i: (0, 0)),
            pl.BlockSpec((H, H), lambda i: (0, 0)),
        ],
        out_specs=[
            pl.BlockSpec((BR, H), lambda i: (i, 0)),
            pl.BlockSpec((BR, H), lambda i: (i, 0)),
        ],
        out_shape=[
            jax.ShapeDtypeStruct((NP, H), _f32),
            jax.ShapeDtypeStruct((NP, H), _f32),
        ],
    )(agg, y_prev, dinv, b, W_next)


def _tc_final_body(a_ref, y_ref, dinv_ref, b_ref, h1_ref, h2_ref,
                   wlin_ref, blin_ref, wg_ref, bg_ref, h_ref, gate_ref):
    agg = a_ref[0] + a_ref[1] + y_ref[...]
    h3 = jnp.maximum(agg * dinv_ref[...] + b_ref[...], 0.0)
    hcat = jnp.concatenate([h1_ref[...], h2_ref[...], h3], axis=1)
    h = jnp.maximum(
        jnp.dot(hcat, wlin_ref[...], preferred_element_type=_f32)
        + blin_ref[...], 0.0)
    h_ref[...] = h
    gate_ref[...] = (
        jnp.dot(h, wg_ref[...], preferred_element_type=_f32) + bg_ref[...])


def _tc_final(agg, y3, dinv, b3, h1, h2, Wlin, blin, Wgate, bgate):
    return pl.pallas_call(
        _tc_final_body,
        grid=(NB,),
        in_specs=[
            pl.BlockSpec((NC, BR, H), lambda i: (0, i, 0)),
            pl.BlockSpec((BR, H), lambda i: (i, 0)),
            pl.BlockSpec((BR, 1), lambda i: (i, 0)),
            pl.BlockSpec((1, H), lambda i: (0, 0)),
            pl.BlockSpec((BR, H), lambda i: (i, 0)),
            pl.BlockSpec((BR, H), lambda i: (i, 0)),
            pl.BlockSpec((3 * H, H), lambda i: (0, 0)),
            pl.BlockSpec((1, H), lambda i: (0, 0)),
            pl.BlockSpec((H, 1), lambda i: (0, 0)),
            pl.BlockSpec((1, 1), lambda i: (0, 0)),
        ],
        out_specs=[
            pl.BlockSpec((BR, H), lambda i: (i, 0)),
            pl.BlockSpec((BR, 1), lambda i: (i, 0)),
        ],
        out_shape=[
            jax.ShapeDtypeStruct((NP, H), _f32),
            jax.ShapeDtypeStruct((NP, 1), _f32),
        ],
    )(agg, y3, dinv, b3, h1, h2, Wlin, blin, Wgate, bgate)


def _tc_pool_body(h_ref, gate_ref, seg_ref, out_ref):
    gate = gate_ref[...]
    rowid = lax.broadcasted_iota(_i32, (NP, 1), 0)
    valid = rowid < N
    m = jnp.max(jnp.where(valid, gate, -jnp.inf))
    a = jnp.where(valid, jnp.exp(gate - m), 0.0)
    onehot = jnp.where(
        seg_ref[...] == lax.broadcasted_iota(_i32, (G, NP), 0),
        1.0, 0.0)
    p = jnp.dot(onehot, a * h_ref[...], preferred_element_type=_f32)
    s = jnp.dot(onehot, a, preferred_element_type=_f32)
    out_ref[...] = p / jnp.maximum(s, 1e-16)


def _tc_pool(h, gate, seg):
    return pl.pallas_call(
        _tc_pool_body,
        out_shape=jax.ShapeDtypeStruct((G, H), _f32),
    )(h, gate, seg)


def kernel(batch, x, edge_index, batch_idx, W1, b1, W2, b2, W3, b3,
           Wlin, blin, Wgate, bgate):
    ei_p = (jnp.full((2, EP), DUMP, _i32)
            .at[:, :E].set(edge_index.astype(_i32)).reshape(2, NCHT, CH))
    xp = jnp.zeros((NP, F_IN), _f32).at[:N].set(x[:, :F_IN])
    seg = jnp.zeros((1, NP), _i32).at[0, :N].set(batch_idx.astype(_i32))

    deg2 = _sc_deg(ei_p).reshape(NC, NP, 1)
    y1, dinv = _tc1(xp, deg2, W1)
    agg1 = _sc_agg(y1, ei_p)
    h1, y2 = _tc_mid(agg1, y1, dinv, b1.reshape(1, H), W2)
    agg2 = _sc_agg(y2, ei_p)
    h2, y3 = _tc_mid(agg2, y2, dinv, b2.reshape(1, H), W3)
    agg3 = _sc_agg(y3, ei_p)
    h, gate = _tc_final(agg3, y3, dinv, b3.reshape(1, H), h1, h2,
                        Wlin, blin.reshape(1, H), Wgate, bgate.reshape(1, 1))
    pooled = _tc_pool(h, gate, seg)
    return h[:N], pooled
